# baseline (speedup 1.0000x reference)
import functools

import jax
import jax.numpy as jnp
from jax import lax
from jax.experimental import pallas as pl
from jax.experimental.pallas import tpu as pltpu
from jax.experimental.pallas import tpu_sc as plsc

N = 10000
C = 128
N_PAD = 10112
ROWS_PT = N_PAD // 16
E_FULL = 320000
RE_FULL = 160000
EP = 323584
REP = 163840
NEG = -1e30


def _pre_body(x_ref, rgx_ref, w1_ref, b1_ref, wa_ref, atta_ref, wr_ref,
              attr_ref, ha_ref, aa_ref, hr_ref, ar_ref):
    w1 = w1_ref[...]
    b1 = b1_ref[...]
    h0 = x_ref[...] @ w1.T + b1
    h0 = jnp.where(h0 >= 0, h0, 0.01 * h0)
    ha = h0 @ wa_ref[...].T
    ha_ref[...] = ha
    aa_ref[...] = ha @ atta_ref[...]
    r0 = rgx_ref[...] @ w1.T + b1
    r0 = jnp.where(r0 >= 0, r0, 0.01 * r0)
    hr = r0 @ wr_ref[...].T
    hr_ref[...] = hr
    ar_ref[...] = hr @ attr_ref[...]


_B = 2000
_G = N // _B

_row = lambda c: pl.BlockSpec((_B, c), lambda i: (i, 0))
_full = lambda *s: pl.BlockSpec(s, lambda i: tuple(0 for _ in s))
_o2 = pl.BlockSpec((2, _B, C), lambda i: (0, i, 0))
_dn = _row(2)


def _pre(x, rgx, w1, b1, wa, atta, wr, attr):
    return pl.pallas_call(
        _pre_body,
        grid=(_G,),
        in_specs=[_row(C), _row(C), _full(C, C), _full(1, C), _full(C, C),
                  _full(C, 2), _full(C, C), _full(C, 2)],
        out_specs=[_row(C), _row(2), _row(C), _row(2)],
        out_shape=[
            jax.ShapeDtypeStruct((N, C), jnp.float32),
            jax.ShapeDtypeStruct((N, 2), jnp.float32),
            jax.ShapeDtypeStruct((N, C), jnp.float32),
            jax.ShapeDtypeStruct((N, 2), jnp.float32),
        ],
    )(x, rgx, w1, b1, wa, atta, wr, attr)


def _combine(o2, dn, a2, h, b):
    t = a2[:, 0:1] + a2[:, 1:2]
    w_self = jnp.exp(jnp.where(t >= 0, t, 0.2 * t))
    num = o2[0] + o2[1] + w_self * h
    den = jnp.sum(dn, axis=1, keepdims=True) + w_self
    return num / den + b


def _postpre_body(o2_ref, dn_ref, aa_ref, ha_ref, ba_ref, wm_ref, attm_ref,
                  hm_ref, am_ref):
    h1 = _combine(o2_ref[...], dn_ref[...], aa_ref[...], ha_ref[...],
                  ba_ref[...])
    hm = h1 @ wm_ref[...].T
    hm_ref[...] = hm
    am_ref[...] = hm @ attm_ref[...]


def _postpre(o2, dn, aa, ha, ba, wm, attm):
    return pl.pallas_call(
        _postpre_body,
        grid=(_G,),
        in_specs=[_o2, _dn, _row(2), _row(C), _full(1, C), _full(C, C),
                  _full(C, 2)],
        out_specs=[_row(C), _row(2)],
        out_shape=[
            jax.ShapeDtypeStruct((N, C), jnp.float32),
            jax.ShapeDtypeStruct((N, 2), jnp.float32),
        ],
    )(o2, dn, aa, ha, ba, wm, attm)


def _final_body(o2m_ref, dnm_ref, am_ref, hm_ref, bm_ref, o2r_ref, dnr_ref,
                ar_ref, hr_ref, br_ref, w2_ref, b2_ref, out_ref):
    out_m = _combine(o2m_ref[...], dnm_ref[...], am_ref[...], hm_ref[...],
                     bm_ref[...])
    rg1 = _combine(o2r_ref[...], dnr_ref[...], ar_ref[...], hr_ref[...],
                   br_ref[...])
    w2 = w2_ref[...]
    out_ref[...] = out_m @ w2[:, :C].T + rg1 @ w2[:, C:].T + b2_ref[...]


def _final(o2m, dnm, am, hm, bm, o2r, dnr, ar, hr, br, w2, b2):
    return pl.pallas_call(
        _final_body,
        grid=(_G,),
        in_specs=[_o2, _dn, _row(2), _row(C), _full(1, C),
                  _o2, _dn, _row(2), _row(C), _full(1, C),
                  _full(C, 2 * C), _full(1, C)],
        out_specs=_row(C),
        out_shape=jax.ShapeDtypeStruct((N, C), jnp.float32),
    )(o2m, dnm, am, hm, bm, o2r, dnr, ar, hr, br, w2, b2)


def _make_gat_scatter(ep):
    per_tile = ep // 32
    n_chunks = per_tile // 128
    mesh = plsc.VectorSubcoreMesh(core_axis_name="c", subcore_axis_name="s")

    @functools.partial(
        pl.kernel,
        out_type=(
            jax.ShapeDtypeStruct((2, N_PAD, C), jnp.float32),
            jax.ShapeDtypeStruct((2 * N_PAD,), jnp.float32),
        ),
        mesh=mesh,
        scratch_types=[
            pltpu.VMEM((128,), jnp.int32),
            pltpu.VMEM((128,), jnp.int32),
            pltpu.VMEM((128,), jnp.int32),
            pltpu.VMEM((128,), jnp.int32),
            pltpu.VMEM((128, C), jnp.float32),
            pltpu.VMEM((128, C), jnp.float32),
            pltpu.VMEM((128,), jnp.float32),
            pltpu.VMEM((128,), jnp.float32),
            pltpu.VMEM((128,), jnp.float32),
            pltpu.VMEM((128,), jnp.float32),
            pltpu.VMEM((128,), jnp.float32),
            pltpu.VMEM((128,), jnp.float32),
            pltpu.VMEM((ROWS_PT,), jnp.float32),
            pltpu.VMEM_SHARED((N_PAD, C), jnp.float32),
            pltpu.VMEM_SHARED((N_PAD,), jnp.float32),
            pltpu.SemaphoreType.DMA,
            pltpu.SemaphoreType.DMA,
            pltpu.SemaphoreType.DMA,
            pltpu.SemaphoreType.DMA,
        ],
        compiler_params=pltpu.CompilerParams(needs_layout_passes=False),
    )
    def gat_scatter(h_hbm, asrc_hbm, adst_hbm, src_hbm, dst_hbm, z2_hbm,
                    z1_hbm, out_hbm, den_hbm, sidx0, didx0, sidx1, didx1,
                    rows0, rows1, asb0, asb1, adb0, adb1, wchunk0, wchunk1,
                    dbuf, out_sh, den_sh, gsem0, gsem1, osem0, osem1):
        cid = lax.axis_index("c")
        sid = lax.axis_index("s")
        wid = cid * 16 + sid
        row0 = sid * ROWS_PT

        pltpu.sync_copy(z2_hbm.at[pl.ds(row0, ROWS_PT)],
                        out_sh.at[pl.ds(row0, ROWS_PT)])

        @pl.when(sid == 0)
        def _():
            pltpu.sync_copy(z1_hbm, den_sh)

        plsc.subcore_barrier()

        base = wid * per_tile

        def load_idx(i, sidx, didx):
            cb = base + i * 128
            pltpu.sync_copy(src_hbm.at[pl.ds(cb, 128)], sidx)
            pltpu.sync_copy(dst_hbm.at[pl.ds(cb, 128)], didx)

        def start_fetch(sidx, didx, rows, asb, adb, sem):
            pltpu.async_copy(h_hbm.at[sidx], rows, sem)
            pltpu.async_copy(asrc_hbm.at[sidx], asb, sem)
            pltpu.async_copy(adst_hbm.at[didx], adb, sem)

        def wait_fetch(sidx, didx, rows, asb, adb, sem):
            pltpu.make_async_copy(h_hbm.at[sidx], rows, sem).wait()
            pltpu.make_async_copy(asrc_hbm.at[sidx], asb, sem).wait()
            pltpu.make_async_copy(adst_hbm.at[didx], adb, sem).wait()

        def compute_scatter(sidx, didx, rows, asb, adb, wchunk, osem):
            for g in range(8):
                d16 = didx[pl.ds(g * 16, 16)]
                a_s = asb[pl.ds(g * 16, 16)]
                a_d = adb[pl.ds(g * 16, 16)]
                t = a_s + a_d
                w = jnp.exp(jnp.where(t >= 0, t, 0.2 * t))
                wchunk[pl.ds(g * 16, 16)] = w
                for j in range(16):
                    cj = lax.gather(
                        w, jnp.full((16, 1), j, jnp.int32),
                        lax.GatherDimensionNumbers(
                            offset_dims=(), collapsed_slice_dims=(0,),
                            start_index_map=(0,)),
                        (1,), mode=lax.GatherScatterMode.PROMISE_IN_BOUNDS)
                    r = g * 16 + j
                    for k in range(8):
                        sl = pl.ds(k * 16, 16)
                        rows[r, sl] = rows[r, sl] * cj
            pltpu.async_copy(rows, out_sh.at[didx], osem, add=True)
            pltpu.async_copy(wchunk, den_sh.at[didx], osem, add=True)

        def wait_scatter(didx, rows, wchunk, osem):
            pltpu.make_async_copy(rows, out_sh.at[didx], osem).wait()
            pltpu.make_async_copy(wchunk, den_sh.at[didx], osem).wait()

        load_idx(0, sidx0, didx0)
        start_fetch(sidx0, didx0, rows0, asb0, adb0, gsem0)

        def pair_body(k, carry):
            b = 2 * k + 1
            a2 = 2 * k + 2

            wait_fetch(sidx0, didx0, rows0, asb0, adb0, gsem0)
            compute_scatter(sidx0, didx0, rows0, asb0, adb0, wchunk0, osem0)

            @pl.when(k > 0)
            def _():
                wait_scatter(didx1, rows1, wchunk1, osem1)

            @pl.when(b < n_chunks)
            def _():
                load_idx(b, sidx1, didx1)
                start_fetch(sidx1, didx1, rows1, asb1, adb1, gsem1)

            @pl.when(b < n_chunks)
            def _():
                wait_fetch(sidx1, didx1, rows1, asb1, adb1, gsem1)
                compute_scatter(sidx1, didx1, rows1, asb1, adb1, wchunk1,
                                osem1)
                wait_scatter(didx0, rows0, wchunk0, osem0)

                @pl.when(a2 < n_chunks)
                def _():
                    load_idx(a2, sidx0, didx0)
                    start_fetch(sidx0, didx0, rows0, asb0, adb0, gsem0)
            return carry

        lax.fori_loop(0, (n_chunks + 1) // 2, pair_body, 0)

        if n_chunks % 2 == 1:
            wait_scatter(didx0, rows0, wchunk0, osem0)
        else:
            wait_scatter(didx1, rows1, wchunk1, osem1)

        plsc.subcore_barrier()
        pltpu.sync_copy(out_sh.at[pl.ds(row0, ROWS_PT)],
                        out_hbm.at[cid, pl.ds(row0, ROWS_PT)])
        pltpu.sync_copy(den_sh.at[pl.ds(row0, ROWS_PT)], dbuf)
        pltpu.sync_copy(dbuf, den_hbm.at[pl.ds(cid * N_PAD + row0, ROWS_PT)])

    return gat_scatter


_gat_scatter_e = _make_gat_scatter(EP)
_gat_scatter_re = _make_gat_scatter(REP)


def _pad_nodes(h, a2):
    h_ext = jnp.concatenate([h, jnp.zeros((N_PAD - N, C), jnp.float32)], 0)
    neg = jnp.full((N_PAD - N,), NEG, jnp.float32)
    a_s = jnp.concatenate([a2[:, 0], neg])
    a_d = jnp.concatenate([a2[:, 1], neg])
    return h_ext, a_s, a_d


def kernel(x, edge_index, edge_attr, rg_x, rg_edge_index, rg_edge_attr,
           W1, b1, Wa, att_src_a, att_dst_a, ba,
           Wr, att_src_r, att_dst_r, br,
           Wm, att_src_m, att_dst_m, bm, W2, b2):
    att_a = jnp.stack([att_src_a, att_dst_a], axis=1)
    att_r = jnp.stack([att_src_r, att_dst_r], axis=1)
    att_m = jnp.stack([att_src_m, att_dst_m], axis=1)

    pad_e = jnp.full((EP - E_FULL,), N, jnp.int32)
    srcp = jnp.concatenate([edge_index[0].astype(jnp.int32), pad_e])
    dstp = jnp.concatenate([edge_index[1].astype(jnp.int32), pad_e])
    pad_r = jnp.full((REP - RE_FULL,), N, jnp.int32)
    rsrcp = jnp.concatenate([rg_edge_index[0].astype(jnp.int32), pad_r])
    rdstp = jnp.concatenate([rg_edge_index[1].astype(jnp.int32), pad_r])

    z2 = jnp.zeros((N_PAD, C), jnp.float32)
    z1 = jnp.zeros((N_PAD,), jnp.float32)

    ha, aa, hr, ar = _pre(x, rg_x, W1, b1[None, :], Wa, att_a, Wr, att_r)

    ha_ext, asrc_a, adst_a = _pad_nodes(ha, aa)
    o2a, dna = _gat_scatter_e(ha_ext, asrc_a, adst_a, srcp, dstp, z2, z1)
    dna = dna.reshape(2, N_PAD)

    hr_ext, asrc_r, adst_r = _pad_nodes(hr, ar)
    o2r, dnr = _gat_scatter_re(hr_ext, asrc_r, adst_r, rsrcp, rdstp, z2, z1)
    dnr = dnr.reshape(2, N_PAD)

    hm, am = _postpre(o2a[:, :N], dna[:, :N].T, aa, ha, ba[None, :], Wm, att_m)

    hm_ext, asrc_m, adst_m = _pad_nodes(hm, am)
    o2m, dnm = _gat_scatter_e(hm_ext, asrc_m, adst_m, srcp, dstp, z2, z1)
    dnm = dnm.reshape(2, N_PAD)

    return _final(o2m[:, :N], dnm[:, :N].T, am, hm, bm[None, :],
                  o2r[:, :N], dnr[:, :N].T, ar, hr, br[None, :],
                  W2, b2[None, :])


# speedup vs baseline: 1.1840x; 1.1840x over previous
import functools

import jax
import jax.numpy as jnp
from jax import lax
from jax.experimental import pallas as pl
from jax.experimental.pallas import tpu as pltpu
from jax.experimental.pallas import tpu_sc as plsc

N = 10000
C = 128
N_PAD = 10112
ROWS_PT = N_PAD // 16
E_FULL = 320000
RE_FULL = 160000
EP = 323584
REP = 163840
NEG = -1e30


def _pre_body(x_ref, rgx_ref, w1_ref, b1_ref, wa_ref, atta_ref, wr_ref,
              attr_ref, ha_ref, aa_ref, hr_ref, ar_ref):
    w1 = w1_ref[...]
    b1 = b1_ref[...]
    h0 = x_ref[...] @ w1.T + b1
    h0 = jnp.where(h0 >= 0, h0, 0.01 * h0)
    ha = h0 @ wa_ref[...].T
    ha_ref[...] = ha
    aa_ref[...] = ha @ atta_ref[...]
    r0 = rgx_ref[...] @ w1.T + b1
    r0 = jnp.where(r0 >= 0, r0, 0.01 * r0)
    hr = r0 @ wr_ref[...].T
    hr_ref[...] = hr
    ar_ref[...] = hr @ attr_ref[...]


_B = 2000
_G = N // _B

_row = lambda c: pl.BlockSpec((_B, c), lambda i: (i, 0))
_full = lambda *s: pl.BlockSpec(s, lambda i: tuple(0 for _ in s))
_o2 = pl.BlockSpec((2, _B, C), lambda i: (0, i, 0))
_dn = _row(2)


def _pre(x, rgx, w1, b1, wa, atta, wr, attr):
    return pl.pallas_call(
        _pre_body,
        grid=(_G,),
        in_specs=[_row(C), _row(C), _full(C, C), _full(1, C), _full(C, C),
                  _full(C, 2), _full(C, C), _full(C, 2)],
        out_specs=[_row(C), _row(2), _row(C), _row(2)],
        out_shape=[
            jax.ShapeDtypeStruct((N, C), jnp.float32),
            jax.ShapeDtypeStruct((N, 2), jnp.float32),
            jax.ShapeDtypeStruct((N, C), jnp.float32),
            jax.ShapeDtypeStruct((N, 2), jnp.float32),
        ],
    )(x, rgx, w1, b1, wa, atta, wr, attr)


def _combine(o2, dn, a2, h, b):
    t = a2[:, 0:1] + a2[:, 1:2]
    w_self = jnp.exp(jnp.where(t >= 0, t, 0.2 * t))
    num = o2[0] + o2[1] + w_self * h
    den = jnp.sum(dn, axis=1, keepdims=True) + w_self
    return num / den + b


def _postpre_body(o2_ref, dn_ref, aa_ref, ha_ref, ba_ref, wm_ref, attm_ref,
                  hm_ref, am_ref):
    h1 = _combine(o2_ref[...], dn_ref[...], aa_ref[...], ha_ref[...],
                  ba_ref[...])
    hm = h1 @ wm_ref[...].T
    hm_ref[...] = hm
    am_ref[...] = hm @ attm_ref[...]


def _postpre(o2, dn, aa, ha, ba, wm, attm):
    return pl.pallas_call(
        _postpre_body,
        grid=(_G,),
        in_specs=[_o2, _dn, _row(2), _row(C), _full(1, C), _full(C, C),
                  _full(C, 2)],
        out_specs=[_row(C), _row(2)],
        out_shape=[
            jax.ShapeDtypeStruct((N, C), jnp.float32),
            jax.ShapeDtypeStruct((N, 2), jnp.float32),
        ],
    )(o2, dn, aa, ha, ba, wm, attm)


def _final_body(o2m_ref, dnm_ref, am_ref, hm_ref, bm_ref, o2r_ref, dnr_ref,
                ar_ref, hr_ref, br_ref, w2_ref, b2_ref, out_ref):
    out_m = _combine(o2m_ref[...], dnm_ref[...], am_ref[...], hm_ref[...],
                     bm_ref[...])
    rg1 = _combine(o2r_ref[...], dnr_ref[...], ar_ref[...], hr_ref[...],
                   br_ref[...])
    w2 = w2_ref[...]
    out_ref[...] = out_m @ w2[:, :C].T + rg1 @ w2[:, C:].T + b2_ref[...]


def _final(o2m, dnm, am, hm, bm, o2r, dnr, ar, hr, br, w2, b2):
    return pl.pallas_call(
        _final_body,
        grid=(_G,),
        in_specs=[_o2, _dn, _row(2), _row(C), _full(1, C),
                  _o2, _dn, _row(2), _row(C), _full(1, C),
                  _full(C, 2 * C), _full(1, C)],
        out_specs=_row(C),
        out_shape=jax.ShapeDtypeStruct((N, C), jnp.float32),
    )(o2m, dnm, am, hm, bm, o2r, dnr, ar, hr, br, w2, b2)


def _make_gat_scatter(ep):
    per_tile = ep // 32
    n_chunks = per_tile // 128
    mesh = plsc.VectorSubcoreMesh(core_axis_name="c", subcore_axis_name="s")

    @functools.partial(
        pl.kernel,
        out_type=(
            jax.ShapeDtypeStruct((2, N_PAD, C), jnp.float32),
            jax.ShapeDtypeStruct((2 * N_PAD,), jnp.float32),
        ),
        mesh=mesh,
        scratch_types=[
            pltpu.VMEM((128,), jnp.int32),
            pltpu.VMEM((128,), jnp.int32),
            pltpu.VMEM((128,), jnp.int32),
            pltpu.VMEM((128,), jnp.int32),
            pltpu.VMEM((128, C), jnp.float32),
            pltpu.VMEM((128, C), jnp.float32),
            pltpu.VMEM((128,), jnp.float32),
            pltpu.VMEM((128,), jnp.float32),
            pltpu.VMEM((128,), jnp.float32),
            pltpu.VMEM((128,), jnp.float32),
            pltpu.VMEM((128,), jnp.float32),
            pltpu.VMEM((128,), jnp.float32),
            pltpu.VMEM((ROWS_PT,), jnp.float32),
            pltpu.VMEM_SHARED((N_PAD, C), jnp.float32),
            pltpu.VMEM_SHARED((N_PAD,), jnp.float32),
            pltpu.SemaphoreType.DMA,
            pltpu.SemaphoreType.DMA,
            pltpu.SemaphoreType.DMA,
            pltpu.SemaphoreType.DMA,
        ],
        compiler_params=pltpu.CompilerParams(needs_layout_passes=False),
    )
    def gat_scatter(h_hbm, asrc_hbm, adst_hbm, src_hbm, dst_hbm, z2_hbm,
                    z1_hbm, out_hbm, den_hbm, sidx0, didx0, sidx1, didx1,
                    rows0, rows1, asb0, asb1, adb0, adb1, wchunk0, wchunk1,
                    dbuf, out_sh, den_sh, gsem0, gsem1, osem0, osem1):
        cid = lax.axis_index("c")
        sid = lax.axis_index("s")
        wid = cid * 16 + sid
        row0 = sid * ROWS_PT

        pltpu.sync_copy(z2_hbm.at[pl.ds(row0, ROWS_PT)],
                        out_sh.at[pl.ds(row0, ROWS_PT)])

        @pl.when(sid == 0)
        def _():
            pltpu.sync_copy(z1_hbm, den_sh)

        plsc.subcore_barrier()

        base = wid * per_tile

        def load_idx(i, sidx, didx):
            cb = base + i * 128
            pltpu.sync_copy(src_hbm.at[pl.ds(cb, 128)], sidx)
            pltpu.sync_copy(dst_hbm.at[pl.ds(cb, 128)], didx)

        def start_fetch(sidx, didx, rows, asb, adb, sem):
            pltpu.async_copy(h_hbm.at[sidx], rows, sem)
            pltpu.async_copy(asrc_hbm.at[sidx], asb, sem)
            pltpu.async_copy(adst_hbm.at[didx], adb, sem)

        def wait_fetch(sidx, didx, rows, asb, adb, sem):
            pltpu.make_async_copy(h_hbm.at[sidx], rows, sem).wait()
            pltpu.make_async_copy(asrc_hbm.at[sidx], asb, sem).wait()
            pltpu.make_async_copy(adst_hbm.at[didx], adb, sem).wait()

        def compute_scatter(sidx, didx, rows, asb, adb, wchunk, osem):
            for g in range(8):
                d16 = didx[pl.ds(g * 16, 16)]
                a_s = asb[pl.ds(g * 16, 16)]
                a_d = adb[pl.ds(g * 16, 16)]
                t = a_s + a_d
                w = jnp.exp(jnp.where(t >= 0, t, 0.2 * t))
                wchunk[pl.ds(g * 16, 16)] = w
                for j in range(16):
                    cj = lax.gather(
                        w, jnp.full((16, 1), j, jnp.int32),
                        lax.GatherDimensionNumbers(
                            offset_dims=(), collapsed_slice_dims=(0,),
                            start_index_map=(0,)),
                        (1,), mode=lax.GatherScatterMode.PROMISE_IN_BOUNDS)
                    r = g * 16 + j
                    for k in range(8):
                        sl = pl.ds(k * 16, 16)
                        rows[r, sl] = rows[r, sl] * cj
            pltpu.sync_copy(rows, out_sh.at[didx], add=True)
            pltpu.sync_copy(wchunk, den_sh.at[didx], add=True)

        load_idx(0, sidx0, didx0)
        start_fetch(sidx0, didx0, rows0, asb0, adb0, gsem0)

        def pair_body(k, carry):
            b = 2 * k + 1

            @pl.when(b < n_chunks)
            def _():
                load_idx(b, sidx1, didx1)
            wait_fetch(sidx0, didx0, rows0, asb0, adb0, gsem0)

            @pl.when(b < n_chunks)
            def _():
                start_fetch(sidx1, didx1, rows1, asb1, adb1, gsem1)
            compute_scatter(sidx0, didx0, rows0, asb0, adb0, wchunk0, osem0)

            @pl.when(b < n_chunks)
            def _():
                a2 = 2 * k + 2

                @pl.when(a2 < n_chunks)
                def _():
                    load_idx(a2, sidx0, didx0)
                wait_fetch(sidx1, didx1, rows1, asb1, adb1, gsem1)

                @pl.when(a2 < n_chunks)
                def _():
                    start_fetch(sidx0, didx0, rows0, asb0, adb0, gsem0)
                compute_scatter(sidx1, didx1, rows1, asb1, adb1, wchunk1,
                                osem1)
            return carry

        lax.fori_loop(0, (n_chunks + 1) // 2, pair_body, 0)

        plsc.subcore_barrier()
        pltpu.sync_copy(out_sh.at[pl.ds(row0, ROWS_PT)],
                        out_hbm.at[cid, pl.ds(row0, ROWS_PT)])
        pltpu.sync_copy(den_sh.at[pl.ds(row0, ROWS_PT)], dbuf)
        pltpu.sync_copy(dbuf, den_hbm.at[pl.ds(cid * N_PAD + row0, ROWS_PT)])

    return gat_scatter


_gat_scatter_e = _make_gat_scatter(EP)
_gat_scatter_re = _make_gat_scatter(REP)


def _pad_nodes(h, a2):
    h_ext = jnp.concatenate([h, jnp.zeros((N_PAD - N, C), jnp.float32)], 0)
    neg = jnp.full((N_PAD - N,), NEG, jnp.float32)
    a_s = jnp.concatenate([a2[:, 0], neg])
    a_d = jnp.concatenate([a2[:, 1], neg])
    return h_ext, a_s, a_d


def kernel(x, edge_index, edge_attr, rg_x, rg_edge_index, rg_edge_attr,
           W1, b1, Wa, att_src_a, att_dst_a, ba,
           Wr, att_src_r, att_dst_r, br,
           Wm, att_src_m, att_dst_m, bm, W2, b2):
    att_a = jnp.stack([att_src_a, att_dst_a], axis=1)
    att_r = jnp.stack([att_src_r, att_dst_r], axis=1)
    att_m = jnp.stack([att_src_m, att_dst_m], axis=1)

    pad_e = jnp.full((EP - E_FULL,), N, jnp.int32)
    srcp = jnp.concatenate([edge_index[0].astype(jnp.int32), pad_e])
    dstp = jnp.concatenate([edge_index[1].astype(jnp.int32), pad_e])
    pad_r = jnp.full((REP - RE_FULL,), N, jnp.int32)
    rsrcp = jnp.concatenate([rg_edge_index[0].astype(jnp.int32), pad_r])
    rdstp = jnp.concatenate([rg_edge_index[1].astype(jnp.int32), pad_r])

    z2 = jnp.zeros((N_PAD, C), jnp.float32)
    z1 = jnp.zeros((N_PAD,), jnp.float32)

    ha, aa, hr, ar = _pre(x, rg_x, W1, b1[None, :], Wa, att_a, Wr, att_r)

    ha_ext, asrc_a, adst_a = _pad_nodes(ha, aa)
    o2a, dna = _gat_scatter_e(ha_ext, asrc_a, adst_a, srcp, dstp, z2, z1)
    dna = dna.reshape(2, N_PAD)

    hr_ext, asrc_r, adst_r = _pad_nodes(hr, ar)
    o2r, dnr = _gat_scatter_re(hr_ext, asrc_r, adst_r, rsrcp, rdstp, z2, z1)
    dnr = dnr.reshape(2, N_PAD)

    hm, am = _postpre(o2a[:, :N], dna[:, :N].T, aa, ha, ba[None, :], Wm, att_m)

    hm_ext, asrc_m, adst_m = _pad_nodes(hm, am)
    o2m, dnm = _gat_scatter_e(hm_ext, asrc_m, adst_m, srcp, dstp, z2, z1)
    dnm = dnm.reshape(2, N_PAD)

    return _final(o2m[:, :N], dnm[:, :N].T, am, hm, bm[None, :],
                  o2r[:, :N], dnr[:, :N].T, ar, hr, br[None, :],
                  W2, b2[None, :])
